# baseline (speedup 1.0000x reference)
import functools

import jax
import jax.numpy as jnp
from jax.experimental import pallas as pl
from jax.experimental.pallas import tpu as pltpu

NQ = 8
BLOCK = 2000


def _linear_body(*refs, nq_rows, nsteps):
    xs = refs[:NQ]
    es = refs[NQ:2 * NQ]
    w1_ref, w2_ref, b_ref, out_hbm, ov, osem = refs[2 * NQ:]

    i = pl.program_id(0)
    slot = jax.lax.rem(i, 2)

    def out_copy(step, j, s):
        return pltpu.make_async_copy(
            ov.at[s, j],
            out_hbm.at[pl.ds(j * nq_rows + step * BLOCK, BLOCK), :],
            osem.at[s, j],
        )

    @pl.when(i >= 2)
    def _drain():
        for j in range(NQ):
            out_copy(i - 2, j, slot).wait()

    for j in range(NQ):
        xb = xs[j][0].astype(jnp.bfloat16)
        eb = es[j][0].astype(jnp.bfloat16)
        acc = jnp.dot(xb, w1_ref[...], preferred_element_type=jnp.float32)
        acc += jnp.dot(eb, w2_ref[...], preferred_element_type=jnp.float32)
        ov[slot, j] = acc + b_ref[...]

    for j in range(NQ):
        out_copy(i, j, slot).start()

    @pl.when(i == nsteps - 1)
    def _epilogue():
        @pl.when(i >= 1)
        def _():
            for j in range(NQ):
                out_copy(i - 1, j, jax.lax.rem(i - 1, 2)).wait()
        for j in range(NQ):
            out_copy(i, j, slot).wait()


@functools.partial(jax.jit, static_argnames=())
def kernel(x, pos, edge_index, edge_attr, W, b):
    del pos, edge_index
    n_edges, d_feat = x.shape
    d_edge = edge_attr.shape[1]
    out_ch = W.shape[0]

    w1 = W[:, :d_feat].T.astype(jnp.bfloat16)
    w2 = W[:, d_feat:].T.astype(jnp.bfloat16)
    b2 = b.reshape(1, out_ch)

    nq_rows = n_edges // NQ
    nsteps = nq_rows // BLOCK
    xq = x.reshape(NQ, nq_rows, d_feat)
    eq = edge_attr.reshape(NQ, nq_rows, d_edge)

    def qmap(q):
        return lambda i: (q, i, 0)

    def cmap(i):
        return (0, 0)

    body = functools.partial(_linear_body, nq_rows=nq_rows, nsteps=nsteps)

    return pl.pallas_call(
        body,
        grid=(nsteps,),
        in_specs=[pl.BlockSpec((1, BLOCK, d_feat), qmap(q)) for q in range(NQ)]
        + [pl.BlockSpec((1, BLOCK, d_edge), qmap(q)) for q in range(NQ)]
        + [
            pl.BlockSpec((d_feat, out_ch), cmap),
            pl.BlockSpec((d_edge, out_ch), cmap),
            pl.BlockSpec((1, out_ch), cmap),
        ],
        out_specs=pl.BlockSpec(memory_space=pl.ANY),
        out_shape=jax.ShapeDtypeStruct((n_edges, out_ch), jnp.float32),
        scratch_shapes=[
            pltpu.VMEM((2, NQ, BLOCK, out_ch), jnp.float32),
            pltpu.SemaphoreType.DMA((2, NQ)),
        ],
    )(*([xq] * NQ), *([eq] * NQ), w1, w2, b2)


# speedup vs baseline: 1.0014x; 1.0014x over previous
import functools

import jax
import jax.numpy as jnp
from jax.experimental import pallas as pl
from jax.experimental.pallas import tpu as pltpu

NQ = 4
BLOCK = 5000


def _linear_body(*refs, nq_rows, nsteps):
    xs = refs[:NQ]
    es = refs[NQ:2 * NQ]
    w1_ref, w2_ref, b_ref, out_hbm, ov, osem = refs[2 * NQ:]

    i = pl.program_id(0)
    slot = jax.lax.rem(i, 2)

    def out_copy(step, j, s):
        return pltpu.make_async_copy(
            ov.at[s, j],
            out_hbm.at[pl.ds(j * nq_rows + step * BLOCK, BLOCK), :],
            osem.at[s, j],
        )

    @pl.when(i >= 2)
    def _drain():
        for j in range(NQ):
            out_copy(i - 2, j, slot).wait()

    for j in range(NQ):
        xb = xs[j][0].astype(jnp.bfloat16)
        eb = es[j][0].astype(jnp.bfloat16)
        acc = jnp.dot(xb, w1_ref[...], preferred_element_type=jnp.float32)
        acc += jnp.dot(eb, w2_ref[...], preferred_element_type=jnp.float32)
        ov[slot, j] = acc + b_ref[...]

    for j in range(NQ):
        out_copy(i, j, slot).start()

    @pl.when(i == nsteps - 1)
    def _epilogue():
        @pl.when(i >= 1)
        def _():
            for j in range(NQ):
                out_copy(i - 1, j, jax.lax.rem(i - 1, 2)).wait()
        for j in range(NQ):
            out_copy(i, j, slot).wait()


@functools.partial(jax.jit, static_argnames=())
def kernel(x, pos, edge_index, edge_attr, W, b):
    del pos, edge_index
    n_edges, d_feat = x.shape
    d_edge = edge_attr.shape[1]
    out_ch = W.shape[0]

    w1 = W[:, :d_feat].T.astype(jnp.bfloat16)
    w2 = W[:, d_feat:].T.astype(jnp.bfloat16)
    b2 = b.reshape(1, out_ch)

    nq_rows = n_edges // NQ
    nsteps = nq_rows // BLOCK
    xq = x.reshape(NQ, nq_rows, d_feat)
    eq = edge_attr.reshape(NQ, nq_rows, d_edge)

    def qmap(q):
        return lambda i: (q, i, 0)

    def cmap(i):
        return (0, 0)

    body = functools.partial(_linear_body, nq_rows=nq_rows, nsteps=nsteps)

    return pl.pallas_call(
        body,
        grid=(nsteps,),
        in_specs=[pl.BlockSpec((1, BLOCK, d_feat), qmap(q)) for q in range(NQ)]
        + [pl.BlockSpec((1, BLOCK, d_edge), qmap(q)) for q in range(NQ)]
        + [
            pl.BlockSpec((d_feat, out_ch), cmap),
            pl.BlockSpec((d_edge, out_ch), cmap),
            pl.BlockSpec((1, out_ch), cmap),
        ],
        out_specs=pl.BlockSpec(memory_space=pl.ANY),
        out_shape=jax.ShapeDtypeStruct((n_edges, out_ch), jnp.float32),
        compiler_params=pltpu.CompilerParams(
            vmem_limit_bytes=64 * 1024 * 1024,
        ),
        scratch_shapes=[
            pltpu.VMEM((2, NQ, BLOCK, out_ch), jnp.float32),
            pltpu.SemaphoreType.DMA((2, NQ)),
        ],
    )(*([xq] * NQ), *([eq] * NQ), w1, w2, b2)


# speedup vs baseline: 1.0360x; 1.0346x over previous
import functools

import jax
import jax.numpy as jnp
from jax.experimental import pallas as pl
from jax.experimental.pallas import tpu as pltpu

NQ = 4
BLOCK = 4000


def _linear_body(*refs, nq_rows, nsteps):
    xs = refs[:NQ]
    es = refs[NQ:2 * NQ]
    w1_ref, w2_ref, b_ref, out_hbm, ov, osem = refs[2 * NQ:]

    i = pl.program_id(0)
    slot = jax.lax.rem(i, 2)

    def out_copy(step, j, s):
        return pltpu.make_async_copy(
            ov.at[s, j],
            out_hbm.at[pl.ds(j * nq_rows + step * BLOCK, BLOCK), :],
            osem.at[s, j],
        )

    for j in range(NQ):
        @pl.when(i >= 2)
        def _drain(j=j):
            out_copy(i - 2, j, slot).wait()

        xb = xs[j][0].astype(jnp.bfloat16)
        eb = es[j][0].astype(jnp.bfloat16)
        acc = jnp.dot(xb, w1_ref[...], preferred_element_type=jnp.float32)
        acc += jnp.dot(eb, w2_ref[...], preferred_element_type=jnp.float32)
        ov[slot, j] = acc + b_ref[...]
        out_copy(i, j, slot).start()

    @pl.when(i == nsteps - 1)
    def _epilogue():
        @pl.when(i >= 1)
        def _():
            for j in range(NQ):
                out_copy(i - 1, j, jax.lax.rem(i - 1, 2)).wait()
        for j in range(NQ):
            out_copy(i, j, slot).wait()


@functools.partial(jax.jit, static_argnames=())
def kernel(x, pos, edge_index, edge_attr, W, b):
    del pos, edge_index
    n_edges, d_feat = x.shape
    d_edge = edge_attr.shape[1]
    out_ch = W.shape[0]

    w1 = W[:, :d_feat].T.astype(jnp.bfloat16)
    w2 = W[:, d_feat:].T.astype(jnp.bfloat16)
    b2 = b.reshape(1, out_ch)

    nq_rows = n_edges // NQ
    nsteps = nq_rows // BLOCK
    xq = x.reshape(NQ, nq_rows, d_feat)
    eq = edge_attr.reshape(NQ, nq_rows, d_edge)

    def qmap(q):
        return lambda i: (q, i, 0)

    def cmap(i):
        return (0, 0)

    body = functools.partial(_linear_body, nq_rows=nq_rows, nsteps=nsteps)

    return pl.pallas_call(
        body,
        grid=(nsteps,),
        in_specs=[pl.BlockSpec((1, BLOCK, d_feat), qmap(q)) for q in range(NQ)]
        + [pl.BlockSpec((1, BLOCK, d_edge), qmap(q)) for q in range(NQ)]
        + [
            pl.BlockSpec((d_feat, out_ch), cmap),
            pl.BlockSpec((d_edge, out_ch), cmap),
            pl.BlockSpec((1, out_ch), cmap),
        ],
        out_specs=pl.BlockSpec(memory_space=pl.ANY),
        out_shape=jax.ShapeDtypeStruct((n_edges, out_ch), jnp.float32),
        compiler_params=pltpu.CompilerParams(
            vmem_limit_bytes=64 * 1024 * 1024,
        ),
        scratch_shapes=[
            pltpu.VMEM((2, NQ, BLOCK, out_ch), jnp.float32),
            pltpu.SemaphoreType.DMA((2, NQ)),
        ],
    )(*([xq] * NQ), *([eq] * NQ), w1, w2, b2)


# speedup vs baseline: 1.0468x; 1.0104x over previous
import functools

import jax
import jax.numpy as jnp
from jax.experimental import pallas as pl
from jax.experimental.pallas import tpu as pltpu

NQ = 4
BLOCK = 4000


def _linear_body(*refs, nq_rows, nsteps):
    xs = refs[:NQ]
    es = refs[NQ:2 * NQ]
    w1_ref, w2_ref, b_ref, out_hbm, ov, osem = refs[2 * NQ:]

    i = pl.program_id(0)
    slot = jax.lax.rem(i, 3)

    def out_copy(step, j, s):
        return pltpu.make_async_copy(
            ov.at[s, j],
            out_hbm.at[pl.ds(j * nq_rows + step * BLOCK, BLOCK), :],
            osem.at[s, j],
        )

    for j in range(NQ):
        xb = xs[j][0].astype(jnp.bfloat16)
        eb = es[j][0].astype(jnp.bfloat16)
        acc = jnp.dot(xb, w1_ref[...], preferred_element_type=jnp.float32)
        acc += jnp.dot(eb, w2_ref[...], preferred_element_type=jnp.float32)
        ov[slot, j] = acc + b_ref[...]

    for j in range(NQ):
        out_copy(i, j, slot).start()

    @pl.when(i >= 2)
    def _drain():
        for j in range(NQ):
            out_copy(i - 2, j, jax.lax.rem(i - 2, 3)).wait()

    @pl.when(i == nsteps - 1)
    def _epilogue():
        @pl.when(i >= 1)
        def _():
            for j in range(NQ):
                out_copy(i - 1, j, jax.lax.rem(i - 1, 3)).wait()
        for j in range(NQ):
            out_copy(i, j, slot).wait()


@functools.partial(jax.jit, static_argnames=())
def kernel(x, pos, edge_index, edge_attr, W, b):
    del pos, edge_index
    n_edges, d_feat = x.shape
    d_edge = edge_attr.shape[1]
    out_ch = W.shape[0]

    w1 = W[:, :d_feat].T.astype(jnp.bfloat16)
    w2 = W[:, d_feat:].T.astype(jnp.bfloat16)
    b2 = b.reshape(1, out_ch)

    nq_rows = n_edges // NQ
    nsteps = nq_rows // BLOCK
    xq = x.reshape(NQ, nq_rows, d_feat)
    eq = edge_attr.reshape(NQ, nq_rows, d_edge)

    def qmap(q):
        return lambda i: (q, i, 0)

    def cmap(i):
        return (0, 0)

    body = functools.partial(_linear_body, nq_rows=nq_rows, nsteps=nsteps)

    return pl.pallas_call(
        body,
        grid=(nsteps,),
        in_specs=[pl.BlockSpec((1, BLOCK, d_feat), qmap(q)) for q in range(NQ)]
        + [pl.BlockSpec((1, BLOCK, d_edge), qmap(q)) for q in range(NQ)]
        + [
            pl.BlockSpec((d_feat, out_ch), cmap),
            pl.BlockSpec((d_edge, out_ch), cmap),
            pl.BlockSpec((1, out_ch), cmap),
        ],
        out_specs=pl.BlockSpec(memory_space=pl.ANY),
        out_shape=jax.ShapeDtypeStruct((n_edges, out_ch), jnp.float32),
        compiler_params=pltpu.CompilerParams(
            vmem_limit_bytes=64 * 1024 * 1024,
        ),
        scratch_shapes=[
            pltpu.VMEM((3, NQ, BLOCK, out_ch), jnp.float32),
            pltpu.SemaphoreType.DMA((3, NQ)),
        ],
    )(*([xq] * NQ), *([eq] * NQ), w1, w2, b2)
